# baseline (speedup 1.0000x reference)
import functools

import jax
import jax.numpy as jnp
from jax import lax
from jax.experimental import pallas as pl
from jax.experimental.pallas import tpu as pltpu
from jax.experimental.pallas import tpu_sc as plsc

_B, _N, _D = 8, 2048, 785
_M = _B * _N
_NW = 32
_RPW = _M // _NW
_CHUNK = 16
_NCHUNK = _RPW // _CHUNK
_NSLC = 49


def _sc_body(cp_h, rp_h, l_h, u_h, lo_h, uo_h, plb_h, pub_h,
             cp_v, rp_v, lbuf, ubuf, plbv, pubv, redv):
    nc = 2
    wid = lax.axis_index("s") * nc + lax.axis_index("c")
    base_row = wid * _RPW

    pltpu.sync_copy(cp_h, cp_v)
    pltpu.sync_copy(rp_h, rp_v)
    zero16 = jnp.zeros((16,), jnp.float32)
    lane = lax.iota(jnp.int32, 16)
    bfly_idx = [(lane ^ sh, lane ^ sh + 16, lane ^ sh + 32, lane ^ sh + 48)
                for sh in (8, 4, 2, 1)]

    def _allreduce4(v0, v1, v2, v3):
        for i0, i1, i2, i3 in bfly_idx:
            redv[pl.ds(0, 16)] = v0
            redv[pl.ds(16, 16)] = v1
            redv[pl.ds(32, 16)] = v2
            redv[pl.ds(48, 16)] = v3
            v0 = v0 + plsc.load_gather(redv, [i0])
            v1 = v1 + plsc.load_gather(redv, [i1])
            v2 = v2 + plsc.load_gather(redv, [i2])
            v3 = v3 + plsc.load_gather(redv, [i3])
        return v0, v1, v2, v3

    b_idx = wid // 4
    n_base = (wid % 4) * _RPW

    def chunk_body(k, _):
        n0 = n_base + k * _CHUNK
        pltpu.sync_copy(l_h.at[b_idx, pl.ds(n0, _CHUNK)], lbuf)
        pltpu.sync_copy(u_h.at[b_idx, pl.ds(n0, _CHUNK)], ubuf)

        def row_body(r, carry):
            plb_acc, pub_acc = carry
            acc_sl = zero16
            acc_tl = zero16
            acc_su = zero16
            acc_tu = zero16
            for j in range(_NSLC):
                off = 16 * j
                lv = lbuf[r, pl.ds(off, 16)]
                uv = ubuf[r, pl.ds(off, 16)]
                cj = cp_v[pl.ds(off, 16)]
                rj = rp_v[pl.ds(off, 16)]
                acc_sl = acc_sl + lv * cj
                acc_tl = acc_tl + jnp.abs(lv) * rj
                acc_su = acc_su + uv * cj
                acc_tu = acc_tu + jnp.abs(uv) * rj
            lane15 = lane == 15
            vb_l = lbuf[r, pl.ds(_D - 16, 16)]
            vb_u = ubuf[r, pl.ds(_D - 16, 16)]
            acc_sl = acc_sl + jnp.where(lane15, vb_l, zero16)
            acc_su = acc_su + jnp.where(lane15, vb_u, zero16)
            s_l, t_l, s_u, t_u = _allreduce4(acc_sl, acc_tl, acc_su, acc_tu)

            conc_lb = s_l - t_l
            max_lb = s_l + t_l
            conc_ub = s_u + t_u
            min_ub = s_u - t_u

            zf = jnp.zeros((16,), jnp.float32)
            onef = jnp.ones((16,), jnp.float32)
            inactive = conc_ub <= zf
            unstable = (conc_lb < zf) & (conc_ub > zf)
            mostly_inactive = unstable & (
                (jnp.abs(conc_lb) > jnp.abs(conc_ub)) | (max_lb <= zf))
            mostly_active = unstable & (jnp.abs(conc_lb) <= jnp.abs(conc_ub))

            denom_l = jnp.where(unstable, max_lb - conc_lb, onef)
            a_l = jnp.where(max_lb < zf, zf, max_lb / denom_l)
            scale_l = jnp.where(inactive | mostly_inactive, zf, onef)
            scale_l = jnp.where(mostly_active, a_l, scale_l)

            zero_x = unstable & (min_ub <= zf)
            denom_u = jnp.where(zero_x, conc_ub - min_ub, onef)
            a_u = conc_ub / denom_u
            scale_u = jnp.where(inactive, zf, onef)
            scale_u = jnp.where(zero_x, a_u, scale_u)
            bias_adj = jnp.where(zero_x, a_u * min_ub, zf)

            for j in range(_NSLC):
                off = 16 * j
                lv = lbuf[r, pl.ds(off, 16)]
                uv = ubuf[r, pl.ds(off, 16)]
                lbuf[r, pl.ds(off, 16)] = scale_l * lv
                ubuf[r, pl.ds(off, 16)] = scale_u * uv
            vb_l = lbuf[r, pl.ds(_D - 16, 16)]
            vb_u = ubuf[r, pl.ds(_D - 16, 16)]
            lbuf[r, pl.ds(_D - 16, 16)] = jnp.where(
                lane15, scale_l * vb_l, vb_l)
            ubuf[r, pl.ds(_D - 16, 16)] = jnp.where(
                lane15, scale_u * vb_u - bias_adj, vb_u)

            rm = lane == r
            plb_acc = jnp.where(rm, scale_l * conc_lb, plb_acc)
            pub_acc = jnp.where(rm, scale_u * conc_ub - bias_adj, pub_acc)
            return plb_acc, pub_acc

        plb_acc, pub_acc = lax.fori_loop(
            0, _CHUNK, row_body, (zero16, zero16))
        plbv[pl.ds(k * _CHUNK, _CHUNK)] = plb_acc
        pubv[pl.ds(k * _CHUNK, _CHUNK)] = pub_acc

        pltpu.sync_copy(lbuf, lo_h.at[b_idx, pl.ds(n0, _CHUNK)])
        pltpu.sync_copy(ubuf, uo_h.at[b_idx, pl.ds(n0, _CHUNK)])
        return 0

    lax.fori_loop(0, _NCHUNK, chunk_body, 0)
    pltpu.sync_copy(plbv, plb_h.at[pl.ds(wid * _RPW, _RPW)])
    pltpu.sync_copy(pubv, pub_h.at[pl.ds(wid * _RPW, _RPW)])


_sc_call = functools.partial(
    pl.kernel,
    out_type=[
        jax.ShapeDtypeStruct((_B, _N, _D), jnp.float32),
        jax.ShapeDtypeStruct((_B, _N, _D), jnp.float32),
        jax.ShapeDtypeStruct((_M,), jnp.float32),
        jax.ShapeDtypeStruct((_M,), jnp.float32),
    ],
    mesh=plsc.VectorSubcoreMesh(core_axis_name="c", subcore_axis_name="s"),
    compiler_params=pltpu.CompilerParams(
        needs_layout_passes=False, skip_device_barrier=True),
    scratch_types=[
        pltpu.VMEM((_NSLC * 16,), jnp.float32),
        pltpu.VMEM((_NSLC * 16,), jnp.float32),
        pltpu.VMEM((_CHUNK, _D), jnp.float32),
        pltpu.VMEM((_CHUNK, _D), jnp.float32),
        pltpu.VMEM((_RPW,), jnp.float32),
        pltpu.VMEM((_RPW,), jnp.float32),
        pltpu.VMEM((64,), jnp.float32),
    ],
)(_sc_body)


def kernel(l, u, in_l, in_u):
    B, N, _ = l.shape
    cp = (in_l + in_u) * 0.5
    rp = (in_u - in_l) * 0.5
    l_new, u_new, post_lb, post_ub = _sc_call(cp, rp, l, u)
    return (l_new, u_new, post_lb.reshape(B, N), post_ub.reshape(B, N))


# speedup vs baseline: 1.2101x; 1.2101x over previous
import functools

import jax
import jax.numpy as jnp
from jax import lax
from jax.experimental import pallas as pl
from jax.experimental.pallas import tpu as pltpu
from jax.experimental.pallas import tpu_sc as plsc

_B, _N, _D = 8, 2048, 785
_M = _B * _N
_NW = 32
_RPW = _M // _NW
_CHUNK = 16
_NCHUNK = _RPW // _CHUNK
_NSUPER = _NCHUNK // 2
_NSLC = 49
_WPB = _N // _RPW


def _sc_body(cp_h, rp_h, l_h, u_h, lo_h, uo_h, plb_h, pub_h,
             cp_v, rp_v, la, ua, lb, ub, plbv, pubv, redv,
             s_in_la, s_in_ua, s_in_lb, s_in_ub,
             s_out_la, s_out_ua, s_out_lb, s_out_ub):
    nc = 2
    wid = lax.axis_index("s") * nc + lax.axis_index("c")
    b_idx = wid // _WPB
    n_base = (wid % _WPB) * _RPW

    pltpu.sync_copy(cp_h, cp_v)
    pltpu.sync_copy(rp_h, rp_v)
    zero16 = jnp.zeros((16,), jnp.float32)
    lane = lax.iota(jnp.int32, 16)
    bfly_idx = [(lane ^ sh, lane ^ sh + 16, lane ^ sh + 32, lane ^ sh + 48)
                for sh in (8, 4, 2, 1)]

    def _allreduce4(v0, v1, v2, v3):
        for i0, i1, i2, i3 in bfly_idx:
            redv[pl.ds(0, 16)] = v0
            redv[pl.ds(16, 16)] = v1
            redv[pl.ds(32, 16)] = v2
            redv[pl.ds(48, 16)] = v3
            v0 = v0 + plsc.load_gather(redv, [i0])
            v1 = v1 + plsc.load_gather(redv, [i1])
            v2 = v2 + plsc.load_gather(redv, [i2])
            v3 = v3 + plsc.load_gather(redv, [i3])
        return v0, v1, v2, v3

    def _src(c):
        n0 = n_base + c * _CHUNK
        return (l_h.at[b_idx, pl.ds(n0, _CHUNK)],
                u_h.at[b_idx, pl.ds(n0, _CHUNK)])

    def _dst(c):
        n0 = n_base + c * _CHUNK
        return (lo_h.at[b_idx, pl.ds(n0, _CHUNK)],
                uo_h.at[b_idx, pl.ds(n0, _CHUNK)])

    def _issue_in(lv, uv, sl, su, c):
        srl, sru = _src(c)
        pltpu.async_copy(srl, lv, sl)
        pltpu.async_copy(sru, uv, su)

    def _wait_in(lv, uv, sl, su, c):
        srl, sru = _src(c)
        pltpu.make_async_copy(srl, lv, sl).wait()
        pltpu.make_async_copy(sru, uv, su).wait()

    def _issue_out(lv, uv, sl, su, c):
        dsl, dsu = _dst(c)
        pltpu.async_copy(lv, dsl, sl)
        pltpu.async_copy(uv, dsu, su)

    def _wait_out(lv, uv, sl, su, c):
        dsl, dsu = _dst(c)
        pltpu.make_async_copy(lv, dsl, sl).wait()
        pltpu.make_async_copy(uv, dsu, su).wait()

    def _compute_chunk(lbuf, ubuf, k):
        def row_body(r, carry):
            plb_acc, pub_acc = carry
            acc_sl = zero16
            acc_tl = zero16
            acc_su = zero16
            acc_tu = zero16
            for j in range(_NSLC):
                off = 16 * j
                lv = lbuf[r, pl.ds(off, 16)]
                uv = ubuf[r, pl.ds(off, 16)]
                cj = cp_v[pl.ds(off, 16)]
                rj = rp_v[pl.ds(off, 16)]
                acc_sl = acc_sl + lv * cj
                acc_tl = acc_tl + jnp.abs(lv) * rj
                acc_su = acc_su + uv * cj
                acc_tu = acc_tu + jnp.abs(uv) * rj
            lane15 = lane == 15
            vb_l = lbuf[r, pl.ds(_D - 16, 16)]
            vb_u = ubuf[r, pl.ds(_D - 16, 16)]
            acc_sl = acc_sl + jnp.where(lane15, vb_l, zero16)
            acc_su = acc_su + jnp.where(lane15, vb_u, zero16)
            s_l, t_l, s_u, t_u = _allreduce4(acc_sl, acc_tl, acc_su, acc_tu)

            conc_lb = s_l - t_l
            max_lb = s_l + t_l
            conc_ub = s_u + t_u
            min_ub = s_u - t_u

            zf = jnp.zeros((16,), jnp.float32)
            onef = jnp.ones((16,), jnp.float32)
            inactive = conc_ub <= zf
            unstable = (conc_lb < zf) & (conc_ub > zf)
            mostly_inactive = unstable & (
                (jnp.abs(conc_lb) > jnp.abs(conc_ub)) | (max_lb <= zf))
            mostly_active = unstable & (jnp.abs(conc_lb) <= jnp.abs(conc_ub))

            denom_l = jnp.where(unstable, max_lb - conc_lb, onef)
            a_l = jnp.where(max_lb < zf, zf, max_lb / denom_l)
            scale_l = jnp.where(inactive | mostly_inactive, zf, onef)
            scale_l = jnp.where(mostly_active, a_l, scale_l)

            zero_x = unstable & (min_ub <= zf)
            denom_u = jnp.where(zero_x, conc_ub - min_ub, onef)
            a_u = conc_ub / denom_u
            scale_u = jnp.where(inactive, zf, onef)
            scale_u = jnp.where(zero_x, a_u, scale_u)
            bias_adj = jnp.where(zero_x, a_u * min_ub, zf)

            for j in range(_NSLC):
                off = 16 * j
                lv = lbuf[r, pl.ds(off, 16)]
                uv = ubuf[r, pl.ds(off, 16)]
                lbuf[r, pl.ds(off, 16)] = scale_l * lv
                ubuf[r, pl.ds(off, 16)] = scale_u * uv
            vb_l = lbuf[r, pl.ds(_D - 16, 16)]
            vb_u = ubuf[r, pl.ds(_D - 16, 16)]
            lbuf[r, pl.ds(_D - 16, 16)] = jnp.where(
                lane15, scale_l * vb_l, vb_l)
            ubuf[r, pl.ds(_D - 16, 16)] = jnp.where(
                lane15, scale_u * vb_u - bias_adj, vb_u)

            rm = lane == r
            plb_acc = jnp.where(rm, scale_l * conc_lb, plb_acc)
            pub_acc = jnp.where(rm, scale_u * conc_ub - bias_adj, pub_acc)
            return plb_acc, pub_acc

        plb_acc, pub_acc = lax.fori_loop(
            0, _CHUNK, row_body, (zero16, zero16))
        plbv[pl.ds(k * _CHUNK, _CHUNK)] = plb_acc
        pubv[pl.ds(k * _CHUNK, _CHUNK)] = pub_acc

    _issue_in(la, ua, s_in_la, s_in_ua, 0)
    _issue_in(lb, ub, s_in_lb, s_in_ub, 1)

    def super_body(i, _):
        ca = 2 * i
        cb = 2 * i + 1
        _wait_in(la, ua, s_in_la, s_in_ua, ca)
        _compute_chunk(la, ua, ca)
        _issue_out(la, ua, s_out_la, s_out_ua, ca)

        _wait_in(lb, ub, s_in_lb, s_in_ub, cb)
        _compute_chunk(lb, ub, cb)
        _issue_out(lb, ub, s_out_lb, s_out_ub, cb)

        @pl.when(i < _NSUPER - 1)
        def _prefetch():
            _wait_out(la, ua, s_out_la, s_out_ua, ca)
            _issue_in(la, ua, s_in_la, s_in_ua, ca + 2)
            _wait_out(lb, ub, s_out_lb, s_out_ub, cb)
            _issue_in(lb, ub, s_in_lb, s_in_ub, cb + 2)
        return 0

    lax.fori_loop(0, _NSUPER, super_body, 0)
    _wait_out(la, ua, s_out_la, s_out_ua, _NCHUNK - 2)
    _wait_out(lb, ub, s_out_lb, s_out_ub, _NCHUNK - 1)
    pltpu.sync_copy(plbv, plb_h.at[pl.ds(wid * _RPW, _RPW)])
    pltpu.sync_copy(pubv, pub_h.at[pl.ds(wid * _RPW, _RPW)])


_sc_call = functools.partial(
    pl.kernel,
    out_type=[
        jax.ShapeDtypeStruct((_B, _N, _D), jnp.float32),
        jax.ShapeDtypeStruct((_B, _N, _D), jnp.float32),
        jax.ShapeDtypeStruct((_M,), jnp.float32),
        jax.ShapeDtypeStruct((_M,), jnp.float32),
    ],
    mesh=plsc.VectorSubcoreMesh(core_axis_name="c", subcore_axis_name="s"),
    compiler_params=pltpu.CompilerParams(needs_layout_passes=False),
    scratch_types=[
        pltpu.VMEM((_NSLC * 16,), jnp.float32),
        pltpu.VMEM((_NSLC * 16,), jnp.float32),
        pltpu.VMEM((_CHUNK, _D), jnp.float32),
        pltpu.VMEM((_CHUNK, _D), jnp.float32),
        pltpu.VMEM((_CHUNK, _D), jnp.float32),
        pltpu.VMEM((_CHUNK, _D), jnp.float32),
        pltpu.VMEM((_RPW,), jnp.float32),
        pltpu.VMEM((_RPW,), jnp.float32),
        pltpu.VMEM((64,), jnp.float32),
    ] + [pltpu.SemaphoreType.DMA] * 8,
)(_sc_body)


def kernel(l, u, in_l, in_u):
    B, N, _ = l.shape
    cp = (in_l + in_u) * 0.5
    rp = (in_u - in_l) * 0.5
    l_new, u_new, post_lb, post_ub = _sc_call(cp, rp, l, u)
    return (l_new, u_new, post_lb.reshape(B, N), post_ub.reshape(B, N))
